# baseline (speedup 1.0000x reference)
import functools

import jax
import jax.numpy as jnp
from jax import lax
from jax.experimental import pallas as pl
from jax.experimental.pallas import tpu as pltpu
from jax.experimental.pallas import tpu_sc as plsc

N = 64
R = 64
D = 64 * 64
NC = 2
NS = 16
L = 16
NW = NC * NS
D_PER_W = N // NW


def _sc_gather_body(x_hbm, y_hbm, a_hbm, rows_hbm, w_hbm,
                    x_v, y_v, w_v, idx_v, rows_v, sem_g, sem_o):
    c = lax.axis_index("c")
    s = lax.axis_index("s")
    wid = s * NC + c
    d_base = wid * D_PER_W

    pltpu.sync_copy(x_hbm, x_v)
    pltpu.sync_copy(y_hbm, y_v)

    lanes = lax.iota(jnp.int32, L)

    samp = jnp.minimum(d_base + (lanes >> 1), N - 1)
    yv = plsc.load_gather(y_v, [samp])
    ty = yv * 2.0 - 1.0
    yy = 0.5 * ((ty + 1.0) * 62.0)
    y0i = yy.astype(jnp.int32)
    y0c = jnp.clip(y0i, 0, R - 1)
    y1c = jnp.clip(y0i + 1, 0, R - 1)
    rowidx = jnp.where((lanes & 1) == 0, y0c, y1c)
    plsc.store_scatter(idx_v, [lanes & 3], rowidx, mask=lanes < 2 * D_PER_W)

    gather = pltpu.async_copy(a_hbm.at[idx_v], rows_v, sem_g)

    @pl.when(wid == 0)
    def _():
        for k in range(N // L):
            sl = pl.ds(k * L, L)
            xk = x_v[sl]
            yk = y_v[sl]
            txk = xk * 2.0 - 1.0
            xx = 0.5 * ((txk + 1.0) * 62.0)
            x0i = xx.astype(jnp.int32)
            x0f = jnp.clip(x0i, 0, R - 1).astype(jnp.float32)
            x1f = jnp.clip(x0i + 1, 0, R - 1).astype(jnp.float32)
            xfac = (x1f - xx) + (xx - x0f)
            tyk = yk * 2.0 - 1.0
            yyk = 0.5 * ((tyk + 1.0) * 62.0)
            ky0 = yyk.astype(jnp.int32)
            ky0f = jnp.clip(ky0, 0, R - 1).astype(jnp.float32)
            ky1f = jnp.clip(ky0 + 1, 0, R - 1).astype(jnp.float32)
            w_v[0, sl] = xfac * (ky1f - yyk)
            w_v[1, sl] = xfac * (yyk - ky0f)
        pltpu.sync_copy(w_v, w_hbm)

    gather.wait()
    h0 = pltpu.async_copy(rows_v.at[pl.ds(0, 2)], rows_hbm.at[d_base], sem_o)
    h1 = pltpu.async_copy(rows_v.at[pl.ds(2, 2)], rows_hbm.at[d_base + 1],
                          sem_o)
    h0.wait()
    h1.wait()


@jax.jit
def _alpha_grid(x_col, y_col, a2d):
    mesh = plsc.VectorSubcoreMesh(core_axis_name="c", subcore_axis_name="s",
                                  num_cores=NC, num_subcores=NS)
    sc_gather = pl.kernel(
        _sc_gather_body,
        out_type=(jax.ShapeDtypeStruct((N, 2, D), jnp.float32),
                  jax.ShapeDtypeStruct((2, N), jnp.float32)),
        mesh=mesh,
        scratch_types=[
            pltpu.VMEM((N,), jnp.float32),
            pltpu.VMEM((N,), jnp.float32),
            pltpu.VMEM((2, N), jnp.float32),
            pltpu.VMEM((2 * D_PER_W,), jnp.int32),
            pltpu.VMEM((2 * D_PER_W, D), jnp.float32),
            pltpu.SemaphoreType.DMA,
            pltpu.SemaphoreType.DMA,
        ],
        compiler_params=pltpu.CompilerParams(needs_layout_passes=False),
    )
    rows2, w = sc_gather(x_col, y_col, a2d)

    def tc_dense(w_ref, rows_ref, out_ref):
        r0 = rows_ref[0, 0, :]
        r1 = rows_ref[0, 1, :]
        for n in range(N):
            out_ref[pl.ds(n * D, D)] = w_ref[0, n] * r0 + w_ref[1, n] * r1

    out = pl.pallas_call(
        tc_dense,
        grid=(N,),
        in_specs=[
            pl.BlockSpec(memory_space=pltpu.SMEM),
            pl.BlockSpec((1, 2, D), lambda i: (i, 0, 0)),
        ],
        out_specs=pl.BlockSpec((N * D,), lambda i: (i,)),
        out_shape=jax.ShapeDtypeStruct((N * N * D,), jnp.float32),
    )(w, rows2)
    return out


def kernel(xyz_sampled, alpha_volume):
    x_col = xyz_sampled[:, 0]
    y_col = xyz_sampled[:, 1]
    a2d = alpha_volume.reshape(R, D)
    return _alpha_grid(x_col, y_col, a2d)


# speedup vs baseline: 1.4302x; 1.4302x over previous
import functools

import jax
import jax.numpy as jnp
from jax import lax
from jax.experimental import pallas as pl
from jax.experimental.pallas import tpu as pltpu
from jax.experimental.pallas import tpu_sc as plsc

N = 64
R = 64
D = 64 * 64
NC = 2
NS = 16
L = 16
NW = NC * NS
D_PER_W = N // NW


def _sc_gather_body(x_hbm, y_hbm, a_hbm, rows_hbm, w_hbm,
                    x_v, y_v, w_v, idx_v, rows_v, sem_g, sem_o):
    c = lax.axis_index("c")
    s = lax.axis_index("s")
    wid = s * NC + c
    d_base = wid * D_PER_W

    pltpu.sync_copy(x_hbm, x_v)
    pltpu.sync_copy(y_hbm, y_v)

    lanes = lax.iota(jnp.int32, L)

    samp = jnp.minimum(d_base + (lanes >> 1), N - 1)
    yv = plsc.load_gather(y_v, [samp])
    ty = yv * 2.0 - 1.0
    yy = 0.5 * ((ty + 1.0) * 62.0)
    y0i = yy.astype(jnp.int32)
    y0c = jnp.clip(y0i, 0, R - 1)
    y1c = jnp.clip(y0i + 1, 0, R - 1)
    rowidx = jnp.where((lanes & 1) == 0, y0c, y1c)
    plsc.store_scatter(idx_v, [lanes & 3], rowidx, mask=lanes < 2 * D_PER_W)

    gather = pltpu.async_copy(a_hbm.at[idx_v], rows_v, sem_g)

    @pl.when(wid == 0)
    def _():
        for k in range(N // L):
            sl = pl.ds(k * L, L)
            xk = x_v[sl]
            yk = y_v[sl]
            txk = xk * 2.0 - 1.0
            xx = 0.5 * ((txk + 1.0) * 62.0)
            x0i = xx.astype(jnp.int32)
            x0f = jnp.clip(x0i, 0, R - 1).astype(jnp.float32)
            x1f = jnp.clip(x0i + 1, 0, R - 1).astype(jnp.float32)
            xfac = (x1f - xx) + (xx - x0f)
            tyk = yk * 2.0 - 1.0
            yyk = 0.5 * ((tyk + 1.0) * 62.0)
            ky0 = yyk.astype(jnp.int32)
            ky0f = jnp.clip(ky0, 0, R - 1).astype(jnp.float32)
            ky1f = jnp.clip(ky0 + 1, 0, R - 1).astype(jnp.float32)
            w_v[0, sl] = xfac * (ky1f - yyk)
            w_v[1, sl] = xfac * (yyk - ky0f)
        pltpu.sync_copy(w_v, w_hbm)

    gather.wait()
    h0 = pltpu.async_copy(rows_v.at[pl.ds(0, 2)], rows_hbm.at[d_base], sem_o)
    h1 = pltpu.async_copy(rows_v.at[pl.ds(2, 2)], rows_hbm.at[d_base + 1],
                          sem_o)
    h0.wait()
    h1.wait()


@jax.jit
def _alpha_grid(x_col, y_col, a2d):
    mesh = plsc.VectorSubcoreMesh(core_axis_name="c", subcore_axis_name="s",
                                  num_cores=NC, num_subcores=NS)
    sc_gather = pl.kernel(
        _sc_gather_body,
        out_type=(jax.ShapeDtypeStruct((N, 2, D), jnp.float32),
                  jax.ShapeDtypeStruct((2, N), jnp.float32)),
        mesh=mesh,
        scratch_types=[
            pltpu.VMEM((N,), jnp.float32),
            pltpu.VMEM((N,), jnp.float32),
            pltpu.VMEM((2, N), jnp.float32),
            pltpu.VMEM((2 * D_PER_W,), jnp.int32),
            pltpu.VMEM((2 * D_PER_W, D), jnp.float32),
            pltpu.SemaphoreType.DMA,
            pltpu.SemaphoreType.DMA,
        ],
        compiler_params=pltpu.CompilerParams(needs_layout_passes=False),
    )
    rows2, w = sc_gather(x_col, y_col, a2d)

    DG = 4

    def tc_dense(w_ref, rows_ref, out_ref):
        for g in range(DG):
            r0 = rows_ref[g, 0, :]
            r1 = rows_ref[g, 1, :]
            for n in range(N):
                out_ref[pl.ds((g * N + n) * D, D)] = (
                    w_ref[0, n] * r0 + w_ref[1, n] * r1)

    out = pl.pallas_call(
        tc_dense,
        grid=(N // DG,),
        in_specs=[
            pl.BlockSpec(memory_space=pltpu.SMEM),
            pl.BlockSpec((DG, 2, D), lambda i: (i, 0, 0)),
        ],
        out_specs=pl.BlockSpec((DG * N * D,), lambda i: (i,)),
        out_shape=jax.ShapeDtypeStruct((N * N * D,), jnp.float32),
    )(w, rows2)
    return out


def kernel(xyz_sampled, alpha_volume):
    x_col = xyz_sampled[:, 0]
    y_col = xyz_sampled[:, 1]
    a2d = alpha_volume.reshape(R, D)
    return _alpha_grid(x_col, y_col, a2d)


# speedup vs baseline: 1.4833x; 1.0371x over previous
import functools

import jax
import jax.numpy as jnp
from jax import lax
from jax.experimental import pallas as pl
from jax.experimental.pallas import tpu as pltpu
from jax.experimental.pallas import tpu_sc as plsc

N = 64
R = 64
D = 64 * 64
NC = 2
NS = 16
L = 16
NW = NC * NS
D_PER_W = N // NW


def _sc_gather_body(x_hbm, y_hbm, a_hbm, rows_hbm, w_hbm,
                    x_v, y_v, w_v, idx_v, rows_v, sem_g, sem_o):
    c = lax.axis_index("c")
    s = lax.axis_index("s")
    wid = s * NC + c
    d_base = wid * D_PER_W

    pltpu.sync_copy(x_hbm, x_v)
    pltpu.sync_copy(y_hbm, y_v)

    lanes = lax.iota(jnp.int32, L)

    samp = jnp.minimum(d_base + (lanes >> 1), N - 1)
    yv = plsc.load_gather(y_v, [samp])
    ty = yv * 2.0 - 1.0
    yy = 0.5 * ((ty + 1.0) * 62.0)
    y0i = yy.astype(jnp.int32)
    y0c = jnp.clip(y0i, 0, R - 1)
    y1c = jnp.clip(y0i + 1, 0, R - 1)
    rowidx = jnp.where((lanes & 1) == 0, y0c, y1c)
    plsc.store_scatter(idx_v, [lanes & 3], rowidx, mask=lanes < 2 * D_PER_W)

    gather = pltpu.async_copy(a_hbm.at[idx_v], rows_v, sem_g)

    @pl.when(wid == 0)
    def _():
        for k in range(N // L):
            sl = pl.ds(k * L, L)
            xk = x_v[sl]
            yk = y_v[sl]
            txk = xk * 2.0 - 1.0
            xx = 0.5 * ((txk + 1.0) * 62.0)
            x0i = xx.astype(jnp.int32)
            x0f = jnp.clip(x0i, 0, R - 1).astype(jnp.float32)
            x1f = jnp.clip(x0i + 1, 0, R - 1).astype(jnp.float32)
            xfac = (x1f - xx) + (xx - x0f)
            tyk = yk * 2.0 - 1.0
            yyk = 0.5 * ((tyk + 1.0) * 62.0)
            ky0 = yyk.astype(jnp.int32)
            ky0f = jnp.clip(ky0, 0, R - 1).astype(jnp.float32)
            ky1f = jnp.clip(ky0 + 1, 0, R - 1).astype(jnp.float32)
            w_v[0, sl] = xfac * (ky1f - yyk)
            w_v[1, sl] = xfac * (yyk - ky0f)
        pltpu.sync_copy(w_v, w_hbm)

    gather.wait()
    h0 = pltpu.async_copy(rows_v.at[pl.ds(0, 2)], rows_hbm.at[d_base], sem_o)
    h1 = pltpu.async_copy(rows_v.at[pl.ds(2, 2)], rows_hbm.at[d_base + 1],
                          sem_o)
    h0.wait()
    h1.wait()


@jax.jit
def _alpha_grid(x_col, y_col, a2d):
    mesh = plsc.VectorSubcoreMesh(core_axis_name="c", subcore_axis_name="s",
                                  num_cores=NC, num_subcores=NS)
    sc_gather = pl.kernel(
        _sc_gather_body,
        out_type=(jax.ShapeDtypeStruct((N, 2, D), jnp.float32),
                  jax.ShapeDtypeStruct((2, N), jnp.float32)),
        mesh=mesh,
        scratch_types=[
            pltpu.VMEM((N,), jnp.float32),
            pltpu.VMEM((N,), jnp.float32),
            pltpu.VMEM((2, N), jnp.float32),
            pltpu.VMEM((2 * D_PER_W,), jnp.int32),
            pltpu.VMEM((2 * D_PER_W, D), jnp.float32),
            pltpu.SemaphoreType.DMA,
            pltpu.SemaphoreType.DMA,
        ],
        compiler_params=pltpu.CompilerParams(needs_layout_passes=False),
    )
    rows2, w = sc_gather(x_col, y_col, a2d)

    DG = 8

    def tc_dense(w_ref, rows_ref, out_ref):
        for g in range(DG):
            r0 = rows_ref[g, 0, :]
            r1 = rows_ref[g, 1, :]
            for n in range(N):
                out_ref[pl.ds((g * N + n) * D, D)] = (
                    w_ref[0, n] * r0 + w_ref[1, n] * r1)

    out = pl.pallas_call(
        tc_dense,
        grid=(N // DG,),
        in_specs=[
            pl.BlockSpec(memory_space=pltpu.SMEM),
            pl.BlockSpec((DG, 2, D), lambda i: (i, 0, 0)),
        ],
        out_specs=pl.BlockSpec((DG * N * D,), lambda i: (i,)),
        out_shape=jax.ShapeDtypeStruct((N * N * D,), jnp.float32),
    )(w, rows2)
    return out


def kernel(xyz_sampled, alpha_volume):
    x_col = xyz_sampled[:, 0]
    y_col = xyz_sampled[:, 1]
    a2d = alpha_volume.reshape(R, D)
    return _alpha_grid(x_col, y_col, a2d)
